# baseline (speedup 1.0000x reference)
import functools

import jax
import jax.numpy as jnp
from jax import lax
from jax.experimental import pallas as pl
from jax.experimental.pallas import tpu as pltpu
from jax.experimental.pallas import tpu_sc as plsc


def _make_sc_kernel(R, H, L, C):
    info = plsc.get_sparse_core_info()
    NC, NS, LANES = info.num_cores, info.num_subcores, info.num_lanes
    NW = NC * NS
    assert R % (NW * C) == 0
    rows_per_w = R // NW
    n_chunks = rows_per_w // C
    assert n_chunks % 2 == 0
    mesh = plsc.VectorSubcoreMesh(core_axis_name="c", subcore_axis_name="s")

    @functools.partial(
        pl.kernel,
        mesh=mesh,
        out_type=jax.ShapeDtypeStruct((R, H), jnp.float32),
        scratch_types=[
            pltpu.VMEM((n_chunks, C * L), jnp.int32),
            pltpu.VMEM((C * L, H), jnp.float32),
            pltpu.VMEM((C * L, H), jnp.float32),
            pltpu.VMEM((C, H), jnp.float32),
            pltpu.SemaphoreType.DMA,
            pltpu.SemaphoreType.DMA,
        ],
    )
    def k(idx_hbm, tables_hbm, out_hbm, idx_v, rows0, rows1, acc_v, sem0, sem1):
        wid = lax.axis_index("s") * NC + lax.axis_index("c")
        base = wid * rows_per_w
        rows = (rows0, rows1)
        sems = (sem0, sem1)

        pltpu.sync_copy(idx_hbm.at[pl.ds(wid * n_chunks, n_chunks)], idx_v)

        def fire(ci, buf):
            pltpu.async_copy(tables_hbm.at[idx_v.at[ci]], rows[buf], sems[buf])

        def drain(ci, buf):
            pltpu.make_async_copy(
                tables_hbm.at[idx_v.at[ci]], rows[buf], sems[buf]
            ).wait()

        def consume(ci, buf):
            rv = rows[buf]

            def h_body(hi, carry):
                off = pl.multiple_of(hi * LANES, LANES)
                for c in range(C):
                    acc = rv[c * L, pl.ds(off, LANES)]
                    for l in range(1, L):
                        acc = acc + rv[c * L + l, pl.ds(off, LANES)]
                    acc_v[c, pl.ds(off, LANES)] = acc
                return carry

            lax.fori_loop(0, H // LANES, h_body, 0)
            pltpu.sync_copy(acc_v, out_hbm.at[pl.ds(base + ci * C, C)])

        fire(0, 0)

        def pair(cj, carry):
            ci0 = cj * 2
            fire(ci0 + 1, 1)
            drain(ci0, 0)
            consume(ci0, 0)

            @pl.when(ci0 + 2 < n_chunks)
            def _():
                fire(ci0 + 2, 0)

            drain(ci0 + 1, 1)
            consume(ci0 + 1, 1)
            return carry

        lax.fori_loop(0, n_chunks // 2, pair, 0)

    return k


def kernel(input_ids, tables):
    B, L, S = input_ids.shape
    _, V, H = tables.shape
    R = B * S
    C = 4
    ids32 = input_ids.astype(jnp.int32)
    flat_idx = ids32 + (jnp.arange(L, dtype=jnp.int32) * V)[None, :, None]
    flat_idx = flat_idx.transpose(0, 2, 1).reshape(R // C, C * L)
    tab = tables.reshape(L * V, H)
    out = _make_sc_kernel(R, H, L, C)(flat_idx, tab)
    return out.reshape(B, S, H)
